# baseline (device time: 124593 ns/iter reference)
import jax
import jax.numpy as jnp
from jax import lax
from jax.experimental import pallas as pl
from jax.experimental.pallas import tpu as pltpu

N_DEV = 4
B_LOC = 2
SQ = 512
SKV = 512
D_MODEL = 768
HG = 8
DH = 64
HD_LOC = HG * DH
H2 = HD_LOC // 2
SH = SQ // 2
BLK = 64


def kernel(x, Wq, K_ext, V_ext, Wo):
    my = lax.axis_index("i")

    K2 = K_ext.reshape(N_DEV * B_LOC, SKV, N_DEV * HG * DH)
    V2 = V_ext.reshape(N_DEV * B_LOC, SKV, N_DEV * HG * DH)
    Wq = Wq.astype(jnp.bfloat16)
    Wo = Wo.astype(jnp.bfloat16)

    def body(x_ref, wq_ref, k_hbm, v_hbm, wo_ref, out_ref,
             kt_ref, vt_ref, slab_k, slab_v,
             wq_comm, wo_comm, wq_send, wq_recv, wo_send, wo_recv,
             slab_sem, head_sem):
        me = lax.axis_index("i")
        left = (me + N_DEV - 1) % N_DEV
        right = (me + 1) % N_DEV

        b0 = me * B_LOC
        slab_copies = [
            pltpu.make_async_copy(k_hbm.at[pl.ds(b0, B_LOC)], slab_k,
                                  slab_sem),
            pltpu.make_async_copy(v_hbm.at[pl.ds(b0, B_LOC)], slab_v,
                                  slab_sem),
        ]
        for c in slab_copies:
            c.start()

        ri = lax.broadcasted_iota(jnp.int32, (SH, SH), 0) // BLK
        ci = lax.broadcasted_iota(jnp.int32, (SH, SH), 1) // BLK
        bias = jnp.where(ci <= ri, 0.0, -1e9).astype(jnp.float32)
        ones_col = jnp.ones((SH, 1), jnp.bfloat16)

        x16 = [x_ref[b, :, :].astype(jnp.bfloat16) for b in range(B_LOC)]

        def attn_head(q, k, v):
            q0, q1 = q[:SH, :], q[SH:, :]
            k0, k1 = k[:SH, :], k[SH:, :]
            vv0 = jnp.concatenate([v[:SH, :], ones_col], axis=1)
            vv1 = jnp.concatenate([v[SH:, :], ones_col], axis=1)
            dn = (((1,), (1,)), ((), ()))
            s00 = lax.dot_general(q0, k0, dn,
                                  preferred_element_type=jnp.float32)
            s10 = lax.dot_general(q1, k0, dn,
                                  preferred_element_type=jnp.float32)
            s11 = lax.dot_general(q1, k1, dn,
                                  preferred_element_type=jnp.float32)
            p00 = jnp.exp(s00 + bias).astype(jnp.bfloat16)
            p10 = jnp.exp(s10).astype(jnp.bfloat16)
            p11 = jnp.exp(s11 + bias).astype(jnp.bfloat16)
            c0 = jnp.dot(p00, vv0, preferred_element_type=jnp.float32)
            c1 = (jnp.dot(p10, vv0, preferred_element_type=jnp.float32)
                  + jnp.dot(p11, vv1, preferred_element_type=jnp.float32))
            ce = jnp.concatenate([c0, c1], axis=0)
            return (ce[:, :DH] / ce[:, DH:DH + 1]).astype(jnp.bfloat16)

        def compute_heads(gh0, nh, wq_c, wo_c, first):
            for b in range(B_LOC):
                Q = (jnp.dot(x16[b], wq_c,
                             preferred_element_type=jnp.float32)
                     * 0.125).astype(jnp.bfloat16)
                cs = []
                for h in range(nh):
                    hp = gh0 // 2 + h // 2
                    lo = (h % 2) * DH
                    q = Q[:, h * DH:(h + 1) * DH]
                    kp = kt_ref[pl.ds(hp, 1), b, :, :].reshape(SKV, 2 * DH)
                    vp = vt_ref[pl.ds(hp, 1), b, :, :].reshape(SKV, 2 * DH)
                    k = kp[:, lo:lo + DH].astype(jnp.bfloat16)
                    v = vp[:, lo:lo + DH].astype(jnp.bfloat16)
                    cs.append(attn_head(q, k, v))
                C = jnp.concatenate(cs, axis=1)
                part = jnp.dot(C, wo_c,
                               preferred_element_type=jnp.float32)
                if first:
                    out_ref[b, :, :] = part
                else:
                    out_ref[b, :, :] = out_ref[b, :, :] + part

        barrier_sem = pltpu.get_barrier_semaphore()
        for nbr in (left, right):
            pl.semaphore_signal(barrier_sem, inc=1, device_id=(nbr,),
                                device_id_type=pl.DeviceIdType.MESH)
        pl.semaphore_wait(barrier_sem, 2)

        def copy(src, dst, ssem, rsem, dev):
            return pltpu.make_async_remote_copy(
                src_ref=src, dst_ref=dst, send_sem=ssem, recv_sem=rsem,
                device_id=(dev,), device_id_type=pl.DeviceIdType.MESH)

        def start_pair(d, src_q, src_o, slot, dev):
            rq = copy(src_q, wq_comm.at[d, slot], wq_send.at[d, slot],
                      wq_recv.at[d, slot], dev)
            ro = copy(src_o, wo_comm.at[d, slot], wo_send.at[d, slot],
                      wo_recv.at[d, slot], dev)
            rq.start()
            ro.start()
            return [rq, ro]

        flights = [[None] * (N_DEV - 1) for _ in range(2)]
        flights[0][0] = start_pair(0, wq_ref.at[:, :H2], wo_ref.at[:H2, :],
                                   0, right)
        flights[1][0] = start_pair(1, wq_ref.at[:, H2:], wo_ref.at[H2:, :],
                                   0, left)

        for c in slab_copies:
            c.wait()
        head_copies = []
        for hp in range(N_DEV * HG // 2):
            for src, dst in ((slab_k, kt_ref), (slab_v, vt_ref)):
                hc = pltpu.make_async_copy(
                    src.at[:, :, pl.ds(hp * 2 * DH, 2 * DH)],
                    dst.at[hp], head_sem)
                hc.start()
                head_copies.append(hc)
        for hc in head_copies:
            hc.wait()

        compute_heads(me * HG, HG, wq_ref[:, :], wo_ref[:, :], first=True)

        for h in range(N_DEV - 1):
            for d in range(2):
                for r in flights[d][h]:
                    r.wait_recv()
                if h < N_DEV - 2:
                    dev = right if d == 0 else left
                    flights[d][h + 1] = start_pair(
                        d, wq_comm.at[d, h], wo_comm.at[d, h], h + 1, dev)
            for d in range(2):
                if d == 0:
                    g = (me + (N_DEV - 1 - h)) % N_DEV
                    gh0 = g * HG
                else:
                    g = (me + h + 1) % N_DEV
                    gh0 = g * HG + H2 // DH
                compute_heads(gh0, H2 // DH, wq_comm[d, h],
                              wo_comm[d, h], first=False)

        for h in range(N_DEV - 1):
            for d in range(2):
                for r in flights[d][h]:
                    r.wait_send()

    return pl.pallas_call(
        body,
        out_shape=jax.ShapeDtypeStruct((B_LOC, SQ, D_MODEL), jnp.float32),
        in_specs=[
            pl.BlockSpec(memory_space=pltpu.VMEM),
            pl.BlockSpec(memory_space=pltpu.VMEM),
            pl.BlockSpec(memory_space=pltpu.MemorySpace.HBM),
            pl.BlockSpec(memory_space=pltpu.MemorySpace.HBM),
            pl.BlockSpec(memory_space=pltpu.VMEM),
        ],
        out_specs=pl.BlockSpec(memory_space=pltpu.VMEM),
        scratch_shapes=[
            pltpu.VMEM((N_DEV * HG // 2, B_LOC, SKV, 2 * DH),
                       jnp.float32),
            pltpu.VMEM((N_DEV * HG // 2, B_LOC, SKV, 2 * DH),
                       jnp.float32),
            pltpu.VMEM((B_LOC, SKV, N_DEV * HG * DH), jnp.float32),
            pltpu.VMEM((B_LOC, SKV, N_DEV * HG * DH), jnp.float32),
            pltpu.VMEM((2, N_DEV - 1, D_MODEL, H2), jnp.bfloat16),
            pltpu.VMEM((2, N_DEV - 1, H2, D_MODEL), jnp.bfloat16),
            pltpu.SemaphoreType.DMA((2, N_DEV - 1)),
            pltpu.SemaphoreType.DMA((2, N_DEV - 1)),
            pltpu.SemaphoreType.DMA((2, N_DEV - 1)),
            pltpu.SemaphoreType.DMA((2, N_DEV - 1)),
            pltpu.SemaphoreType.DMA(()),
            pltpu.SemaphoreType.DMA(()),
        ],
        compiler_params=pltpu.CompilerParams(
            collective_id=0,
            vmem_limit_bytes=100 * 1024 * 1024,
        ),
    )(x, Wq, K2, V2, Wo)


# device time: 74483 ns/iter; 1.6728x vs baseline; 1.6728x over previous
import jax
import jax.numpy as jnp
from jax import lax
from jax.experimental import pallas as pl
from jax.experimental.pallas import tpu as pltpu

N_DEV = 4
B_LOC = 2
SQ = 512
SKV = 512
D_MODEL = 768
HG = 8
DH = 64
HD_LOC = HG * DH
H2 = HD_LOC // 2
SH = SQ // 2
BLK = 64


def kernel(x, Wq, K_ext, V_ext, Wo):
    my = lax.axis_index("i")

    K_t = jnp.transpose(
        lax.dynamic_slice_in_dim(K_ext, my * B_LOC, B_LOC, axis=0
                                 ).astype(jnp.bfloat16), (2, 0, 1, 3))
    V_t = jnp.transpose(
        lax.dynamic_slice_in_dim(V_ext, my * B_LOC, B_LOC, axis=0
                                 ).astype(jnp.bfloat16), (2, 0, 1, 3))

    def body(x_ref, wq_ref, kt_ref, vt_ref, wo_ref, out_ref,
             wq16, wo16,
             wq_comm, wo_comm, wq_send, wq_recv, wo_send, wo_recv):
        me = lax.axis_index("i")
        left = (me + N_DEV - 1) % N_DEV
        right = (me + 1) % N_DEV

        ri = lax.broadcasted_iota(jnp.int32, (SH, SH), 0) // BLK
        ci = lax.broadcasted_iota(jnp.int32, (SH, SH), 1) // BLK
        bias = jnp.where(ci <= ri, 0.0, -1e9).astype(jnp.float32)
        ones_col = jnp.ones((SH, 1), jnp.bfloat16)

        x16 = [x_ref[b, :, :].astype(jnp.bfloat16) for b in range(B_LOC)]
        wq16[:, :] = wq_ref[:, :].astype(jnp.bfloat16)
        wo16[:, :] = wo_ref[:, :].astype(jnp.bfloat16)

        def attn_head(q, k, v):
            q0, q1 = q[:SH, :], q[SH:, :]
            k0, k1 = k[:SH, :], k[SH:, :]
            vv0 = jnp.concatenate([v[:SH, :], ones_col], axis=1)
            vv1 = jnp.concatenate([v[SH:, :], ones_col], axis=1)
            dn = (((1,), (1,)), ((), ()))
            s00 = lax.dot_general(q0, k0, dn,
                                  preferred_element_type=jnp.float32)
            s10 = lax.dot_general(q1, k0, dn,
                                  preferred_element_type=jnp.float32)
            s11 = lax.dot_general(q1, k1, dn,
                                  preferred_element_type=jnp.float32)
            p00 = jnp.exp(s00 + bias).astype(jnp.bfloat16)
            p10 = jnp.exp(s10).astype(jnp.bfloat16)
            p11 = jnp.exp(s11 + bias).astype(jnp.bfloat16)
            c0 = jnp.dot(p00, vv0, preferred_element_type=jnp.float32)
            c1 = (jnp.dot(p10, vv0, preferred_element_type=jnp.float32)
                  + jnp.dot(p11, vv1, preferred_element_type=jnp.float32))
            ce = jnp.concatenate([c0, c1], axis=0)
            return (ce[:, :DH] / ce[:, DH:DH + 1]).astype(jnp.bfloat16)

        def compute_heads(gh0, nh, wq_c, wo_c, first):
            for b in range(B_LOC):
                Q = (jnp.dot(x16[b], wq_c,
                             preferred_element_type=jnp.float32)
                     * 0.125).astype(jnp.bfloat16)
                cs = []
                for h in range(nh):
                    gh = gh0 + h
                    q = Q[:, h * DH:(h + 1) * DH]
                    k = kt_ref[pl.ds(gh, 1), b, :, :].reshape(SKV, DH)
                    v = vt_ref[pl.ds(gh, 1), b, :, :].reshape(SKV, DH)
                    cs.append(attn_head(q, k, v))
                C = jnp.concatenate(cs, axis=1)
                part = jnp.dot(C, wo_c,
                               preferred_element_type=jnp.float32)
                if first:
                    out_ref[b, :, :] = part
                else:
                    out_ref[b, :, :] = out_ref[b, :, :] + part

        barrier_sem = pltpu.get_barrier_semaphore()
        for nbr in (left, right):
            pl.semaphore_signal(barrier_sem, inc=1, device_id=(nbr,),
                                device_id_type=pl.DeviceIdType.MESH)
        pl.semaphore_wait(barrier_sem, 2)

        def copy(src, dst, ssem, rsem, dev):
            return pltpu.make_async_remote_copy(
                src_ref=src, dst_ref=dst, send_sem=ssem, recv_sem=rsem,
                device_id=(dev,), device_id_type=pl.DeviceIdType.MESH)

        def start_pair(d, src_q, src_o, slot, dev):
            rq = copy(src_q, wq_comm.at[d, slot], wq_send.at[d, slot],
                      wq_recv.at[d, slot], dev)
            ro = copy(src_o, wo_comm.at[d, slot], wo_send.at[d, slot],
                      wo_recv.at[d, slot], dev)
            rq.start()
            ro.start()
            return [rq, ro]

        flights = [[None] * (N_DEV - 1) for _ in range(2)]
        flights[0][0] = start_pair(0, wq16.at[:, :H2], wo16.at[:H2, :],
                                   0, right)
        flights[1][0] = start_pair(1, wq16.at[:, H2:], wo16.at[H2:, :],
                                   0, left)

        if True:
            compute_heads(me * HG, HG, wq16[:, :], wo16[:, :],
                          first=True)

        for h in range(N_DEV - 1):
            if True:
                for d in range(2):
                    for r in flights[d][h]:
                        r.wait_recv()
                    if h < N_DEV - 2:
                        dev = right if d == 0 else left
                        flights[d][h + 1] = start_pair(
                            d, wq_comm.at[d, h], wo_comm.at[d, h], h + 1,
                            dev)
            if True:
                for d in range(2):
                    if d == 0:
                        g = (me + (N_DEV - 1 - h)) % N_DEV
                        gh0 = g * HG
                    else:
                        g = (me + h + 1) % N_DEV
                        gh0 = g * HG + H2 // DH
                    compute_heads(gh0, H2 // DH, wq_comm[d, h],
                                  wo_comm[d, h], first=False)

        for h in range(N_DEV - 1):
            for d in range(2):
                for r in flights[d][h]:
                    r.wait_send()

    return pl.pallas_call(
        body,
        out_shape=jax.ShapeDtypeStruct((B_LOC, SQ, D_MODEL), jnp.float32),
        in_specs=[pl.BlockSpec(memory_space=pltpu.VMEM)] * 5,
        out_specs=pl.BlockSpec(memory_space=pltpu.VMEM),
        scratch_shapes=[
            pltpu.VMEM((D_MODEL, HD_LOC), jnp.bfloat16),
            pltpu.VMEM((HD_LOC, D_MODEL), jnp.bfloat16),
            pltpu.VMEM((2, N_DEV - 1, D_MODEL, H2), jnp.bfloat16),
            pltpu.VMEM((2, N_DEV - 1, H2, D_MODEL), jnp.bfloat16),
            pltpu.SemaphoreType.DMA((2, N_DEV - 1)),
            pltpu.SemaphoreType.DMA((2, N_DEV - 1)),
            pltpu.SemaphoreType.DMA((2, N_DEV - 1)),
            pltpu.SemaphoreType.DMA((2, N_DEV - 1)),
        ],
        compiler_params=pltpu.CompilerParams(
            collective_id=0,
            vmem_limit_bytes=100 * 1024 * 1024,
        ),
    )(x, Wq, K_t, V_t, Wo)


# device time: 68396 ns/iter; 1.8216x vs baseline; 1.0890x over previous
import jax
import jax.numpy as jnp
from jax import lax
from jax.experimental import pallas as pl
from jax.experimental.pallas import tpu as pltpu

N_DEV = 4
B_LOC = 2
SQ = 512
SKV = 512
D_MODEL = 768
HG = 8
DH = 64
HD_LOC = HG * DH
H2 = HD_LOC // 2
SH = SQ // 2
BLK = 64


def kernel(x, Wq, K_ext, V_ext, Wo):
    my = lax.axis_index("i")

    def _pairs(a):
        a = a.astype(jnp.bfloat16).reshape(B_LOC, SKV, N_DEV * HG // 2,
                                           2 * DH)
        return jnp.transpose(a, (2, 0, 1, 3))

    K_t = _pairs(lax.dynamic_slice_in_dim(K_ext, my * B_LOC, B_LOC, axis=0))
    V_t = _pairs(lax.dynamic_slice_in_dim(V_ext, my * B_LOC, B_LOC, axis=0))

    def body(x_ref, wq_ref, kt_ref, vt_ref, wo_ref, out_ref,
             wq16, wo16,
             wq_comm, wo_comm, wq_send, wq_recv, wo_send, wo_recv):
        me = lax.axis_index("i")
        left = (me + N_DEV - 1) % N_DEV
        right = (me + 1) % N_DEV

        ri = lax.broadcasted_iota(jnp.int32, (SH, SH), 0) // BLK
        ci = lax.broadcasted_iota(jnp.int32, (SH, SH), 1) // BLK
        bias = jnp.where(ci <= ri, 0.0, -1e9).astype(jnp.float32)
        ones_col = jnp.ones((SH, 1), jnp.bfloat16)

        x16 = [x_ref[b, :, :].astype(jnp.bfloat16) for b in range(B_LOC)]
        wq16[:, :] = wq_ref[:, :].astype(jnp.bfloat16)
        wo16[:, :] = wo_ref[:, :].astype(jnp.bfloat16)

        def attn_head(q, k, v):
            q0, q1 = q[:SH, :], q[SH:, :]
            k0, k1 = k[:SH, :], k[SH:, :]
            vv0 = jnp.concatenate([v[:SH, :], ones_col], axis=1)
            vv1 = jnp.concatenate([v[SH:, :], ones_col], axis=1)
            dn = (((1,), (1,)), ((), ()))
            s00 = lax.dot_general(q0, k0, dn,
                                  preferred_element_type=jnp.float32)
            s10 = lax.dot_general(q1, k0, dn,
                                  preferred_element_type=jnp.float32)
            s11 = lax.dot_general(q1, k1, dn,
                                  preferred_element_type=jnp.float32)
            p00 = jnp.exp(s00 + bias).astype(jnp.bfloat16)
            p10 = jnp.exp(s10).astype(jnp.bfloat16)
            p11 = jnp.exp(s11 + bias).astype(jnp.bfloat16)
            c0 = jnp.dot(p00, vv0, preferred_element_type=jnp.float32)
            c1 = (jnp.dot(p10, vv0, preferred_element_type=jnp.float32)
                  + jnp.dot(p11, vv1, preferred_element_type=jnp.float32))
            ce = jnp.concatenate([c0, c1], axis=0)
            return (ce[:, :DH] / ce[:, DH:DH + 1]).astype(jnp.bfloat16)

        def compute_heads(gh0, nh, wq_c, wo_c, first):
            for b in range(B_LOC):
                Q = (jnp.dot(x16[b], wq_c,
                             preferred_element_type=jnp.float32)
                     * 0.125).astype(jnp.bfloat16)
                cs = []
                for h in range(nh):
                    hp = gh0 // 2 + h // 2
                    lo = (h % 2) * DH
                    q = Q[:, h * DH:(h + 1) * DH]
                    kp = kt_ref[pl.ds(hp, 1), b, :, :].reshape(SKV, 2 * DH)
                    vp = vt_ref[pl.ds(hp, 1), b, :, :].reshape(SKV, 2 * DH)
                    cs.append(attn_head(q, kp[:, lo:lo + DH],
                                        vp[:, lo:lo + DH]))
                C = jnp.concatenate(cs, axis=1)
                part = jnp.dot(C, wo_c,
                               preferred_element_type=jnp.float32)
                if first:
                    out_ref[b, :, :] = part
                else:
                    out_ref[b, :, :] = out_ref[b, :, :] + part

        barrier_sem = pltpu.get_barrier_semaphore()
        for nbr in (left, right):
            pl.semaphore_signal(barrier_sem, inc=1, device_id=(nbr,),
                                device_id_type=pl.DeviceIdType.MESH)
        pl.semaphore_wait(barrier_sem, 2)

        def copy(src, dst, ssem, rsem, dev):
            return pltpu.make_async_remote_copy(
                src_ref=src, dst_ref=dst, send_sem=ssem, recv_sem=rsem,
                device_id=(dev,), device_id_type=pl.DeviceIdType.MESH)

        def start_pair(d, src_q, src_o, slot, dev):
            rq = copy(src_q, wq_comm.at[d, slot], wq_send.at[d, slot],
                      wq_recv.at[d, slot], dev)
            ro = copy(src_o, wo_comm.at[d, slot], wo_send.at[d, slot],
                      wo_recv.at[d, slot], dev)
            rq.start()
            ro.start()
            return [rq, ro]

        flights = [[None] * (N_DEV - 1) for _ in range(2)]
        flights[0][0] = start_pair(0, wq16.at[:, :H2], wo16.at[:H2, :],
                                   0, right)
        flights[1][0] = start_pair(1, wq16.at[:, H2:], wo16.at[H2:, :],
                                   0, left)

        if True:
            compute_heads(me * HG, HG, wq16[:, :], wo16[:, :],
                          first=True)

        for h in range(N_DEV - 1):
            if True:
                for d in range(2):
                    for r in flights[d][h]:
                        r.wait_recv()
                    if h < N_DEV - 2:
                        dev = right if d == 0 else left
                        flights[d][h + 1] = start_pair(
                            d, wq_comm.at[d, h], wo_comm.at[d, h], h + 1,
                            dev)
            if True:
                for d in range(2):
                    if d == 0:
                        g = (me + (N_DEV - 1 - h)) % N_DEV
                        gh0 = g * HG
                    else:
                        g = (me + h + 1) % N_DEV
                        gh0 = g * HG + H2 // DH
                    compute_heads(gh0, H2 // DH, wq_comm[d, h],
                                  wo_comm[d, h], first=False)

        for h in range(N_DEV - 1):
            for d in range(2):
                for r in flights[d][h]:
                    r.wait_send()

    return pl.pallas_call(
        body,
        out_shape=jax.ShapeDtypeStruct((B_LOC, SQ, D_MODEL), jnp.float32),
        in_specs=[pl.BlockSpec(memory_space=pltpu.VMEM)] * 5,
        out_specs=pl.BlockSpec(memory_space=pltpu.VMEM),
        scratch_shapes=[
            pltpu.VMEM((D_MODEL, HD_LOC), jnp.bfloat16),
            pltpu.VMEM((HD_LOC, D_MODEL), jnp.bfloat16),
            pltpu.VMEM((2, N_DEV - 1, D_MODEL, H2), jnp.bfloat16),
            pltpu.VMEM((2, N_DEV - 1, H2, D_MODEL), jnp.bfloat16),
            pltpu.SemaphoreType.DMA((2, N_DEV - 1)),
            pltpu.SemaphoreType.DMA((2, N_DEV - 1)),
            pltpu.SemaphoreType.DMA((2, N_DEV - 1)),
            pltpu.SemaphoreType.DMA((2, N_DEV - 1)),
        ],
        compiler_params=pltpu.CompilerParams(
            collective_id=0,
            vmem_limit_bytes=100 * 1024 * 1024,
        ),
    )(x, Wq, K_t, V_t, Wo)


# device time: 65880 ns/iter; 1.8912x vs baseline; 1.0382x over previous
import jax
import jax.numpy as jnp
from jax import lax
from jax.experimental import pallas as pl
from jax.experimental.pallas import tpu as pltpu

N_DEV = 4
B_LOC = 2
SQ = 512
SKV = 512
D_MODEL = 768
HG = 8
DH = 64
HD_LOC = HG * DH
H2 = HD_LOC // 2
SH = SQ // 2
BLK = 64


def kernel(x, Wq, K_ext, V_ext, Wo):
    my = lax.axis_index("i")

    def _pairs(a):
        a = a.astype(jnp.bfloat16).reshape(B_LOC, SKV, N_DEV * HG // 2,
                                           2 * DH)
        return jnp.transpose(a, (2, 0, 1, 3))

    K_t = _pairs(lax.dynamic_slice_in_dim(K_ext, my * B_LOC, B_LOC, axis=0))
    V_t = _pairs(lax.dynamic_slice_in_dim(V_ext, my * B_LOC, B_LOC, axis=0))

    def body(x_ref, wq_ref, kt_hbm, vt_hbm, wo_ref, out_ref,
             wq16, wo16, kt_ref, vt_ref,
             wq_comm, wo_comm, wq_send, wq_recv, wo_send, wo_recv,
             kv_sem):
        me = lax.axis_index("i")
        left = (me + N_DEV - 1) % N_DEV
        right = (me + 1) % N_DEV

        kv_copies = [pltpu.make_async_copy(kt_hbm, kt_ref, kv_sem),
                     pltpu.make_async_copy(vt_hbm, vt_ref, kv_sem)]
        for c in kv_copies:
            c.start()

        ri = lax.broadcasted_iota(jnp.int32, (SH, SH), 0) // BLK
        ci = lax.broadcasted_iota(jnp.int32, (SH, SH), 1) // BLK
        bias = jnp.where(ci <= ri, 0.0, -1e9).astype(jnp.float32)
        ones_col = jnp.ones((SH, 1), jnp.bfloat16)

        x16 = [x_ref[b, :, :].astype(jnp.bfloat16) for b in range(B_LOC)]
        wq16[:, :] = wq_ref[:, :].astype(jnp.bfloat16)
        wo16[:, :] = wo_ref[:, :].astype(jnp.bfloat16)

        def attn_head(q, k, v):
            q0, q1 = q[:SH, :], q[SH:, :]
            k0, k1 = k[:SH, :], k[SH:, :]
            vv0 = jnp.concatenate([v[:SH, :], ones_col], axis=1)
            vv1 = jnp.concatenate([v[SH:, :], ones_col], axis=1)
            dn = (((1,), (1,)), ((), ()))
            s00 = lax.dot_general(q0, k0, dn,
                                  preferred_element_type=jnp.float32)
            s10 = lax.dot_general(q1, k0, dn,
                                  preferred_element_type=jnp.float32)
            s11 = lax.dot_general(q1, k1, dn,
                                  preferred_element_type=jnp.float32)
            p00 = jnp.exp(s00 + bias).astype(jnp.bfloat16)
            p10 = jnp.exp(s10).astype(jnp.bfloat16)
            p11 = jnp.exp(s11 + bias).astype(jnp.bfloat16)
            c0 = jnp.dot(p00, vv0, preferred_element_type=jnp.float32)
            c1 = (jnp.dot(p10, vv0, preferred_element_type=jnp.float32)
                  + jnp.dot(p11, vv1, preferred_element_type=jnp.float32))
            ce = jnp.concatenate([c0, c1], axis=0)
            return (ce[:, :DH] / ce[:, DH:DH + 1]).astype(jnp.bfloat16)

        def compute_heads(gh0, nh, wq_c, wo_c, first):
            for b in range(B_LOC):
                Q = (jnp.dot(x16[b], wq_c,
                             preferred_element_type=jnp.float32)
                     * 0.125).astype(jnp.bfloat16)
                cs = []
                for h in range(nh):
                    hp = gh0 // 2 + h // 2
                    lo = (h % 2) * DH
                    q = Q[:, h * DH:(h + 1) * DH]
                    kp = kt_ref[pl.ds(hp, 1), b, :, :].reshape(SKV, 2 * DH)
                    vp = vt_ref[pl.ds(hp, 1), b, :, :].reshape(SKV, 2 * DH)
                    cs.append(attn_head(q, kp[:, lo:lo + DH],
                                        vp[:, lo:lo + DH]))
                C = jnp.concatenate(cs, axis=1)
                part = jnp.dot(C, wo_c,
                               preferred_element_type=jnp.float32)
                if first:
                    out_ref[b, :, :] = part
                else:
                    out_ref[b, :, :] = out_ref[b, :, :] + part

        barrier_sem = pltpu.get_barrier_semaphore()
        for nbr in (left, right):
            pl.semaphore_signal(barrier_sem, inc=1, device_id=(nbr,),
                                device_id_type=pl.DeviceIdType.MESH)
        pl.semaphore_wait(barrier_sem, 2)

        def copy(src, dst, ssem, rsem, dev):
            return pltpu.make_async_remote_copy(
                src_ref=src, dst_ref=dst, send_sem=ssem, recv_sem=rsem,
                device_id=(dev,), device_id_type=pl.DeviceIdType.MESH)

        def start_pair(d, src_q, src_o, slot, dev):
            rq = copy(src_q, wq_comm.at[d, slot], wq_send.at[d, slot],
                      wq_recv.at[d, slot], dev)
            ro = copy(src_o, wo_comm.at[d, slot], wo_send.at[d, slot],
                      wo_recv.at[d, slot], dev)
            rq.start()
            ro.start()
            return [rq, ro]

        flights = [[None] * (N_DEV - 1) for _ in range(2)]
        flights[0][0] = start_pair(0, wq16.at[:, :H2], wo16.at[:H2, :],
                                   0, right)
        flights[1][0] = start_pair(1, wq16.at[:, H2:], wo16.at[H2:, :],
                                   0, left)

        for c in kv_copies:
            c.wait()

        if True:
            compute_heads(me * HG, HG, wq16[:, :], wo16[:, :],
                          first=True)

        for h in range(N_DEV - 1):
            if True:
                for d in range(2):
                    for r in flights[d][h]:
                        r.wait_recv()
                    if h < N_DEV - 2:
                        dev = right if d == 0 else left
                        flights[d][h + 1] = start_pair(
                            d, wq_comm.at[d, h], wo_comm.at[d, h], h + 1,
                            dev)
            if True:
                for d in range(2):
                    if d == 0:
                        g = (me + (N_DEV - 1 - h)) % N_DEV
                        gh0 = g * HG
                    else:
                        g = (me + h + 1) % N_DEV
                        gh0 = g * HG + H2 // DH
                    compute_heads(gh0, H2 // DH, wq_comm[d, h],
                                  wo_comm[d, h], first=False)

        for h in range(N_DEV - 1):
            for d in range(2):
                for r in flights[d][h]:
                    r.wait_send()

    return pl.pallas_call(
        body,
        out_shape=jax.ShapeDtypeStruct((B_LOC, SQ, D_MODEL), jnp.float32),
        in_specs=[
            pl.BlockSpec(memory_space=pltpu.VMEM),
            pl.BlockSpec(memory_space=pltpu.VMEM),
            pl.BlockSpec(memory_space=pltpu.MemorySpace.HBM),
            pl.BlockSpec(memory_space=pltpu.MemorySpace.HBM),
            pl.BlockSpec(memory_space=pltpu.VMEM),
        ],
        out_specs=pl.BlockSpec(memory_space=pltpu.VMEM),
        scratch_shapes=[
            pltpu.VMEM((D_MODEL, HD_LOC), jnp.bfloat16),
            pltpu.VMEM((HD_LOC, D_MODEL), jnp.bfloat16),
            pltpu.VMEM((N_DEV * HG // 2, B_LOC, SKV, 2 * DH),
                       jnp.bfloat16),
            pltpu.VMEM((N_DEV * HG // 2, B_LOC, SKV, 2 * DH),
                       jnp.bfloat16),
            pltpu.VMEM((2, N_DEV - 1, D_MODEL, H2), jnp.bfloat16),
            pltpu.VMEM((2, N_DEV - 1, H2, D_MODEL), jnp.bfloat16),
            pltpu.SemaphoreType.DMA((2, N_DEV - 1)),
            pltpu.SemaphoreType.DMA((2, N_DEV - 1)),
            pltpu.SemaphoreType.DMA((2, N_DEV - 1)),
            pltpu.SemaphoreType.DMA((2, N_DEV - 1)),
            pltpu.SemaphoreType.DMA(()),
        ],
        compiler_params=pltpu.CompilerParams(
            collective_id=0,
            vmem_limit_bytes=100 * 1024 * 1024,
        ),
    )(x, Wq, K_t, V_t, Wo)
